# Initial kernel scaffold; baseline (speedup 1.0000x reference)
import jax
import jax.numpy as jnp
from jax.experimental import pallas as pl


def kernel(x, batch, edge_index, pseudo, Wg1, mu1, sigma1, Wroot1, b1, Wg2, mu2, sigma2, Wroot2, b2, Wfc, bfc, Wl1, bl1, Wl2, bl2):
    raise NotImplementedError("write your pallas kernel here")


# speedup vs baseline: 110.9124x; 110.9124x over previous
import functools

import jax
import jax.numpy as jnp
from jax import lax
from jax.experimental import pallas as pl
from jax.experimental.pallas import tpu as pltpu
from jax.experimental.pallas import tpu_sc as plsc

BS = 2
N_NODES = 15135
E = 484320
N_FEAT = 128
HID = 16
K = 2
D = 2
HFC = 256
NCLS = 2
EPS = 1e-15

NP = 16384
BN = 512
NBLK = NP // BN
RPT = NP // 16

NC = 2
NS = 16
CHUNK = 512
CPW = 30
EW = CHUNK * CPW
E_PAD = EW * NC * NS
EROWS = E_PAD // 128


def _tc1_body(x_ref, wg_ref, wr_ref, b_ref, g_ref, root_ref):
    wg = wg_ref[...]
    wr = wr_ref[...]
    b = b_ref[...]
    x0 = x_ref[0]
    x1 = x_ref[1]
    g_ref[:, 0:32] = jnp.dot(x0, wg, preferred_element_type=jnp.float32)
    g_ref[:, 32:64] = jnp.dot(x1, wg, preferred_element_type=jnp.float32)
    root_ref[:, 0:16] = jnp.dot(x0, wr, preferred_element_type=jnp.float32) + b
    root_ref[:, 16:32] = jnp.dot(x1, wr, preferred_element_type=jnp.float32) + b


def _tc1(xp, Wg1, Wroot1, b1):
    return pl.pallas_call(
        _tc1_body,
        grid=(NBLK,),
        in_specs=[
            pl.BlockSpec((BS, BN, N_FEAT), lambda i: (0, i, 0)),
            pl.BlockSpec((N_FEAT, K * HID), lambda i: (0, 0)),
            pl.BlockSpec((N_FEAT, HID), lambda i: (0, 0)),
            pl.BlockSpec((1, HID), lambda i: (0, 0)),
        ],
        out_specs=[
            pl.BlockSpec((BN, 2 * K * HID), lambda i: (i, 0)),
            pl.BlockSpec((BN, 2 * HID), lambda i: (i, 0)),
        ],
        out_shape=[
            jax.ShapeDtypeStruct((NP, 2 * K * HID), jnp.float32),
            jax.ShapeDtypeStruct((NP, 2 * HID), jnp.float32),
        ],
    )(xp, Wg1, Wroot1, b1)


def _sc_layer_body(with_cnt, g_hbm, srcp, dstp, pT, params, z2d, z1d, ones_hbm,
                   *rest):
    if with_cnt:
        (out_acc, out_cnt, acc, cntacc, src_v, dst_v, rows_v, msg_v,
         p0_v, p1_v, w0_v, w1_v, ones_v, params_v) = rest
    else:
        (out_acc, acc, src_v, dst_v, rows_v, msg_v,
         p0_v, p1_v, w0_v, w1_v, ones_v, params_v) = rest
        out_cnt = cntacc = None

    c = lax.axis_index("c")
    s = lax.axis_index("s")
    wid = c * NS + s

    pltpu.sync_copy(z2d, acc.at[pl.ds(s * RPT, RPT)])
    if with_cnt:
        pltpu.sync_copy(z1d, cntacc.at[pl.ds(s * RPT, RPT)])

    pltpu.sync_copy(ones_hbm, ones_v)
    pltpu.sync_copy(params, params_v)

    plsc.subcore_barrier()

    m00 = params_v[pl.ds(0, 16)]
    m01 = params_v[pl.ds(16, 16)]
    m10 = params_v[pl.ds(32, 16)]
    m11 = params_v[pl.ds(48, 16)]
    s00 = params_v[pl.ds(64, 16)]
    s01 = params_v[pl.ds(80, 16)]
    s10 = params_v[pl.ds(96, 16)]
    s11 = params_v[pl.ds(112, 16)]
    c00 = -0.5 / (EPS + s00 * s00)
    c01 = -0.5 / (EPS + s01 * s01)
    c10 = -0.5 / (EPS + s10 * s10)
    c11 = -0.5 / (EPS + s11 * s11)

    def chunk_body(i, carry):
        e_off = wid * EW + i * CHUNK
        r_off = wid * (EW // 128) + i * (CHUNK // 128)
        pltpu.sync_copy(srcp.at[pl.ds(r_off, CHUNK // 128)], src_v)
        pltpu.sync_copy(dstp.at[pl.ds(r_off, CHUNK // 128)], dst_v)
        pltpu.sync_copy(pT.at[0, pl.ds(e_off, CHUNK)], p0_v)
        pltpu.sync_copy(pT.at[1, pl.ds(e_off, CHUNK)], p1_v)

        def wbody(kk, carry2):
            p0 = p0_v[pl.ds(kk * 16, 16)]
            p1 = p1_v[pl.ds(kk * 16, 16)]
            d00 = p0 - m00
            d01 = p1 - m01
            w0_v[pl.ds(kk * 16, 16)] = jnp.exp(d00 * d00 * c00 + d01 * d01 * c01)
            d10 = p0 - m10
            d11 = p1 - m11
            w1_v[pl.ds(kk * 16, 16)] = jnp.exp(d10 * d10 * c10 + d11 * d11 * c11)
            return carry2

        lax.fori_loop(0, CHUNK // 16, wbody, 0)

        for j in range(CHUNK // 128):
            pltpu.sync_copy(g_hbm.at[src_v.at[j]],
                            rows_v.at[pl.ds(j * 128, 128)])

        def ebody(e, carry2):
            bw0 = jnp.full((16,), w0_v[pl.ds(e, 1)][0], jnp.float32)
            bw1 = jnp.full((16,), w1_v[pl.ds(e, 1)][0], jnp.float32)
            r00 = rows_v[e, pl.ds(0, 16)]
            r01 = rows_v[e, pl.ds(16, 16)]
            r10 = rows_v[e, pl.ds(32, 16)]
            r11 = rows_v[e, pl.ds(48, 16)]
            msg_v[e, pl.ds(0, 16)] = r00 * bw0 + r01 * bw1
            msg_v[e, pl.ds(16, 16)] = r10 * bw0 + r11 * bw1
            return carry2

        lax.fori_loop(0, CHUNK, ebody, 0)

        for j in range(CHUNK // 128):
            pltpu.sync_copy(msg_v.at[pl.ds(j * 128, 128)],
                            acc.at[dst_v.at[j]], add=True)
            if with_cnt:
                pltpu.sync_copy(ones_v, cntacc.at[dst_v.at[j]], add=True)
        return carry

    lax.fori_loop(0, CPW, chunk_body, 0)

    plsc.subcore_barrier()

    pltpu.sync_copy(acc.at[pl.ds(s * RPT, RPT)],
                    out_acc.at[c, pl.ds(s * RPT, RPT)])
    if with_cnt:
        pltpu.sync_copy(cntacc.at[pl.ds(s * RPT, RPT)],
                        out_cnt.at[c, 0, pl.ds(s * RPT, RPT)])


def _sc_layer(g_hbm, srcp, dstp, pT, params, with_cnt):
    mesh = plsc.VectorSubcoreMesh(core_axis_name="c", subcore_axis_name="s")
    out_type = [jax.ShapeDtypeStruct((NC, NP, 2 * HID), jnp.float32)]
    scratch = [
        pltpu.VMEM_SHARED((NP, 2 * HID), jnp.float32),
    ]
    if with_cnt:
        out_type.append(jax.ShapeDtypeStruct((NC, 1, NP), jnp.float32))
        scratch.append(pltpu.VMEM_SHARED((NP,), jnp.float32))
    scratch += [
        pltpu.VMEM((CHUNK // 128, 128), jnp.int32),
        pltpu.VMEM((CHUNK // 128, 128), jnp.int32),
        pltpu.VMEM((CHUNK, 4 * HID), jnp.float32),
        pltpu.VMEM((CHUNK, 2 * HID), jnp.float32),
        pltpu.VMEM((CHUNK,), jnp.float32),
        pltpu.VMEM((CHUNK,), jnp.float32),
        pltpu.VMEM((CHUNK,), jnp.float32),
        pltpu.VMEM((CHUNK,), jnp.float32),
        pltpu.VMEM((128,), jnp.float32),
        pltpu.VMEM((128,), jnp.float32),
    ]
    z2d = jnp.zeros((RPT, 2 * HID), jnp.float32)
    z1d = jnp.zeros((RPT,), jnp.float32)
    ones128 = jnp.ones((128,), jnp.float32)
    fn = pl.kernel(
        functools.partial(_sc_layer_body, with_cnt),
        out_type=out_type,
        mesh=mesh,
        scratch_types=scratch,
        compiler_params=pltpu.CompilerParams(use_tc_tiling_on_sc=False),
    )
    return fn(g_hbm, srcp, dstp, pT, params, z2d, z1d, ones128)


def _elu(v):
    return jnp.where(v > 0, v, jnp.exp(v) - 1.0)


def _tc2_body(acc_ref, cnt_ref, root_ref, wg2_ref, wr2_ref, b2_ref,
              h1_ref, g2_ref, root2_ref):
    a = acc_ref[0] + acc_ref[1]
    cc = cnt_ref[0, 0, :] + cnt_ref[1, 0, :]
    inv = 1.0 / jnp.maximum(cc, 1.0)
    pre = a * inv[:, None] + root_ref[...]
    h1 = _elu(pre)
    h1_ref[...] = h1
    wg2 = wg2_ref[...]
    wr2 = wr2_ref[...]
    b2 = b2_ref[...]
    h1b0 = h1[:, 0:16]
    h1b1 = h1[:, 16:32]
    g2_ref[:, 0:32] = jnp.dot(h1b0, wg2, preferred_element_type=jnp.float32)
    g2_ref[:, 32:64] = jnp.dot(h1b1, wg2, preferred_element_type=jnp.float32)
    root2_ref[:, 0:16] = jnp.dot(h1b0, wr2, preferred_element_type=jnp.float32) + b2
    root2_ref[:, 16:32] = jnp.dot(h1b1, wr2, preferred_element_type=jnp.float32) + b2


def _tc2(acc1, cnt1, root1, Wg2, Wroot2, b2):
    return pl.pallas_call(
        _tc2_body,
        grid=(NBLK,),
        in_specs=[
            pl.BlockSpec((NC, BN, 2 * HID), lambda i: (0, i, 0)),
            pl.BlockSpec((NC, 1, BN), lambda i: (0, 0, i)),
            pl.BlockSpec((BN, 2 * HID), lambda i: (i, 0)),
            pl.BlockSpec((HID, K * HID), lambda i: (0, 0)),
            pl.BlockSpec((HID, HID), lambda i: (0, 0)),
            pl.BlockSpec((1, HID), lambda i: (0, 0)),
        ],
        out_specs=[
            pl.BlockSpec((BN, 2 * HID), lambda i: (i, 0)),
            pl.BlockSpec((BN, 2 * K * HID), lambda i: (i, 0)),
            pl.BlockSpec((BN, 2 * HID), lambda i: (i, 0)),
        ],
        out_shape=[
            jax.ShapeDtypeStruct((NP, 2 * HID), jnp.float32),
            jax.ShapeDtypeStruct((NP, 2 * K * HID), jnp.float32),
            jax.ShapeDtypeStruct((NP, 2 * HID), jnp.float32),
        ],
    )(acc1, cnt1, root1, Wg2, Wroot2, b2)


def _tc3_body(acc_ref, cnt_ref, root2_ref, h1_ref, wfce_ref, wfco_ref,
              bfc_ref, wl1_ref, bl1_ref, wl2_ref, bl2_ref, out_ref, zacc_ref):
    i = pl.program_id(0)
    a = acc_ref[0] + acc_ref[1]
    cc = cnt_ref[0, 0, :] + cnt_ref[1, 0, :]
    inv = 1.0 / jnp.maximum(cc, 1.0)
    h2 = _elu(a * inv[:, None] + root2_ref[...])
    h1 = h1_ref[...]
    wfce = wfce_ref[...]
    wfco = wfco_ref[...]
    s0 = (jnp.dot(h1[:, 0:16], wfce, preferred_element_type=jnp.float32)
          + jnp.dot(h2[:, 0:16], wfco, preferred_element_type=jnp.float32))
    s1 = (jnp.dot(h1[:, 16:32], wfce, preferred_element_type=jnp.float32)
          + jnp.dot(h2[:, 16:32], wfco, preferred_element_type=jnp.float32))
    sblk = jnp.concatenate([s0, s1], axis=1) + bfc_ref[...]
    contrib = lax.dot_general(sblk, wl1_ref[...],
                              (((0,), (0,)), ((), ())),
                              preferred_element_type=jnp.float32)

    @pl.when(i == 0)
    def _():
        zacc_ref[...] = jnp.zeros_like(zacc_ref)

    zacc_ref[...] += contrib

    @pl.when(i == NBLK - 1)
    def _():
        z = _elu(zacc_ref[...] + bl1_ref[...])
        zz = jnp.dot(z, wl2_ref[...], preferred_element_type=jnp.float32) + bl2_ref[...]
        m = jnp.max(zz, axis=-1, keepdims=True)
        lse = m + jnp.log(jnp.sum(jnp.exp(zz - m), axis=-1, keepdims=True))
        out_ref[...] = zz - lse


def _tc3(acc2, cnt1, root2, h1, wfce, wfco, bfc, Wl1p, bl1, Wl2, bl2):
    return pl.pallas_call(
        _tc3_body,
        grid=(NBLK,),
        in_specs=[
            pl.BlockSpec((NC, BN, 2 * HID), lambda i: (0, i, 0)),
            pl.BlockSpec((NC, 1, BN), lambda i: (0, 0, i)),
            pl.BlockSpec((BN, 2 * HID), lambda i: (i, 0)),
            pl.BlockSpec((BN, 2 * HID), lambda i: (i, 0)),
            pl.BlockSpec((HID, 1), lambda i: (0, 0)),
            pl.BlockSpec((HID, 1), lambda i: (0, 0)),
            pl.BlockSpec((1, 1), lambda i: (0, 0)),
            pl.BlockSpec((BN, HFC), lambda i: (i, 0)),
            pl.BlockSpec((1, HFC), lambda i: (0, 0)),
            pl.BlockSpec((HFC, NCLS), lambda i: (0, 0)),
            pl.BlockSpec((1, NCLS), lambda i: (0, 0)),
        ],
        out_specs=pl.BlockSpec((BS, NCLS), lambda i: (0, 0)),
        out_shape=jax.ShapeDtypeStruct((BS, NCLS), jnp.float32),
        scratch_shapes=[pltpu.VMEM((BS, HFC), jnp.float32)],
    )(acc2, cnt1, root2, h1, wfce, wfco, bfc, Wl1p, bl1, Wl2, bl2)


def kernel(x, batch, edge_index, pseudo, Wg1, mu1, sigma1, Wroot1, b1,
           Wg2, mu2, sigma2, Wroot2, b2, Wfc, bfc, Wl1, bl1, Wl2, bl2):
    f32 = jnp.float32
    xp = jnp.pad(x, ((0, 0), (0, NP - N_NODES), (0, 0)))
    src = edge_index[0]
    dst = edge_index[1]
    pad_e = E_PAD - E
    srcp = jnp.concatenate([src, jnp.full((pad_e,), N_NODES, jnp.int32)])
    dstp = jnp.concatenate([dst, jnp.full((pad_e,), N_NODES, jnp.int32)])
    srcp = srcp.reshape(EROWS, 128)
    dstp = dstp.reshape(EROWS, 128)
    pT = jnp.concatenate([pseudo.T, jnp.zeros((D, pad_e), f32)], axis=1)

    params1 = jnp.repeat(
        jnp.concatenate([mu1.reshape(-1), sigma1.reshape(-1)]), 16)
    params2 = jnp.repeat(
        jnp.concatenate([mu2.reshape(-1), sigma2.reshape(-1)]), 16)

    g1, root1 = _tc1(xp, Wg1, Wroot1, b1.reshape(1, HID))
    acc1, cnt1 = _sc_layer(g1, srcp, dstp, pT, params1, with_cnt=True)
    h1, g2, root2 = _tc2(acc1, cnt1, root1, Wg2, Wroot2, b2.reshape(1, HID))
    (acc2,) = _sc_layer(g2, srcp, dstp, pT, params2, with_cnt=False)

    wfce = Wfc[0::2, :]
    wfco = Wfc[1::2, :]
    Wl1p = jnp.pad(Wl1, ((0, NP - N_NODES), (0, 0)))
    out = _tc3(acc2, cnt1, root2, h1, wfce, wfco, bfc.reshape(1, 1),
               Wl1p, bl1.reshape(1, HFC), Wl2, bl2.reshape(1, NCLS))
    return out


# speedup vs baseline: 167.6194x; 1.5113x over previous
import functools

import jax
import jax.numpy as jnp
from jax import lax
from jax.experimental import pallas as pl
from jax.experimental.pallas import tpu as pltpu
from jax.experimental.pallas import tpu_sc as plsc

BS = 2
N_NODES = 15135
E = 484320
N_FEAT = 128
HID = 16
K = 2
D = 2
HFC = 256
NCLS = 2
EPS = 1e-15

NP = 16384
BN = 512
NBLK = NP // BN
RPT = NP // 16

NC = 2
NS = 16
CHUNK = 512
CPW = 30
EW = CHUNK * CPW
E_PAD = EW * NC * NS
EROWS = E_PAD // 128
NCHUNKS = E_PAD // CHUNK


def _tc1_body(x_ref, wg_ref, wr_ref, b_ref, g_ref, root_ref):
    wg = wg_ref[...]
    wr = wr_ref[...]
    b = b_ref[...]
    x0 = x_ref[0]
    x1 = x_ref[1]
    g_ref[:, 0:32] = jnp.dot(x0, wg, preferred_element_type=jnp.float32)
    g_ref[:, 32:64] = jnp.dot(x1, wg, preferred_element_type=jnp.float32)
    root_ref[:, 0:16] = jnp.dot(x0, wr, preferred_element_type=jnp.float32) + b
    root_ref[:, 16:32] = jnp.dot(x1, wr, preferred_element_type=jnp.float32) + b


def _tc1(xp, Wg1, Wroot1, b1):
    return pl.pallas_call(
        _tc1_body,
        grid=(NBLK,),
        in_specs=[
            pl.BlockSpec((BS, BN, N_FEAT), lambda i: (0, i, 0)),
            pl.BlockSpec((N_FEAT, K * HID), lambda i: (0, 0)),
            pl.BlockSpec((N_FEAT, HID), lambda i: (0, 0)),
            pl.BlockSpec((1, HID), lambda i: (0, 0)),
        ],
        out_specs=[
            pl.BlockSpec((BN, 2 * K * HID), lambda i: (i, 0)),
            pl.BlockSpec((BN, 2 * HID), lambda i: (i, 0)),
        ],
        out_shape=[
            jax.ShapeDtypeStruct((NP, 2 * K * HID), jnp.float32),
            jax.ShapeDtypeStruct((NP, 2 * HID), jnp.float32),
        ],
    )(xp, Wg1, Wroot1, b1)


def _sc_layer_body(with_cnt, g_hbm, sd_hbm, pp_hbm, params, z2d, z1d, ones_hbm,
                   *rest):
    if with_cnt:
        (out_acc, out_cnt, acc, cntacc, sd_v, pp_v, rows_v, msg_v,
         w0_v, w1_v, ones_v, params_v, gsem) = rest
    else:
        (out_acc, acc, sd_v, pp_v, rows_v, msg_v,
         w0_v, w1_v, ones_v, params_v, gsem) = rest
        out_cnt = cntacc = None

    c = lax.axis_index("c")
    s = lax.axis_index("s")
    wid = c * NS + s

    pltpu.sync_copy(z2d, acc.at[pl.ds(s * RPT, RPT)])
    if with_cnt:
        pltpu.sync_copy(z1d, cntacc.at[pl.ds(s * RPT, RPT)])

    pltpu.sync_copy(ones_hbm, ones_v)
    pltpu.sync_copy(params, params_v)

    plsc.subcore_barrier()

    m00 = params_v[pl.ds(0, 16)]
    m01 = params_v[pl.ds(16, 16)]
    m10 = params_v[pl.ds(32, 16)]
    m11 = params_v[pl.ds(48, 16)]
    s00 = params_v[pl.ds(64, 16)]
    s01 = params_v[pl.ds(80, 16)]
    s10 = params_v[pl.ds(96, 16)]
    s11 = params_v[pl.ds(112, 16)]
    c00 = -0.5 / (EPS + s00 * s00)
    c01 = -0.5 / (EPS + s01 * s01)
    c10 = -0.5 / (EPS + s10 * s10)
    c11 = -0.5 / (EPS + s11 * s11)

    base_r = wid * (EW // 128)
    base_c = wid * CPW

    def load_small(ch, slot):
        pltpu.sync_copy(sd_hbm.at[pl.ds(base_r + ch * (CHUNK // 128),
                                        CHUNK // 128)], sd_v.at[slot])
        pltpu.sync_copy(pp_hbm.at[base_c + ch], pp_v.at[slot])

    def start_gather(slot):
        for j in range(CHUNK // 128):
            pltpu.async_copy(g_hbm.at[sd_v.at[slot, j, 0]],
                             rows_v.at[slot, pl.ds(j * 128, 128)], gsem[slot])

    def process(slot):
        def wbody(kk, carry2):
            p0 = pp_v[slot, 0, pl.ds(kk * 16, 16)]
            p1 = pp_v[slot, 1, pl.ds(kk * 16, 16)]
            d00 = p0 - m00
            d01 = p1 - m01
            w0_v[slot, pl.ds(kk * 16, 16)] = jnp.exp(d00 * d00 * c00 + d01 * d01 * c01)
            d10 = p0 - m10
            d11 = p1 - m11
            w1_v[slot, pl.ds(kk * 16, 16)] = jnp.exp(d10 * d10 * c10 + d11 * d11 * c11)
            return carry2

        lax.fori_loop(0, CHUNK // 16, wbody, 0, unroll=2)

        pltpu.make_async_copy(g_hbm.at[pl.ds(0, CHUNK)],
                              rows_v.at[slot], gsem[slot]).wait()

        def ebody(e, carry2):
            bw0 = jnp.full((16,), w0_v[slot, pl.ds(e, 1)][0], jnp.float32)
            bw1 = jnp.full((16,), w1_v[slot, pl.ds(e, 1)][0], jnp.float32)
            r00 = rows_v[slot, e, pl.ds(0, 16)]
            r01 = rows_v[slot, e, pl.ds(16, 16)]
            r10 = rows_v[slot, e, pl.ds(32, 16)]
            r11 = rows_v[slot, e, pl.ds(48, 16)]
            msg_v[e, pl.ds(0, 16)] = r00 * bw0 + r01 * bw1
            msg_v[e, pl.ds(16, 16)] = r10 * bw0 + r11 * bw1
            return carry2

        lax.fori_loop(0, CHUNK, ebody, 0, unroll=4)

        for j in range(CHUNK // 128):
            pltpu.sync_copy(msg_v.at[pl.ds(j * 128, 128)],
                            acc.at[sd_v.at[slot, j, 1]], add=True)
            if with_cnt:
                pltpu.sync_copy(ones_v, cntacc.at[sd_v.at[slot, j, 1]],
                                add=True)

    load_small(0, 0)
    start_gather(0)

    def pair_body(i, carry):
        c0 = 2 * i
        load_small(c0 + 1, 1)
        start_gather(1)
        process(0)

        @pl.when(i < CPW // 2 - 1)
        def _():
            load_small(c0 + 2, 0)
            start_gather(0)

        process(1)
        return carry

    lax.fori_loop(0, CPW // 2, pair_body, 0)

    plsc.subcore_barrier()

    pltpu.sync_copy(acc.at[pl.ds(s * RPT, RPT)],
                    out_acc.at[c, pl.ds(s * RPT, RPT)])
    if with_cnt:
        pltpu.sync_copy(cntacc.at[pl.ds(s * RPT, RPT)],
                        out_cnt.at[c, 0, pl.ds(s * RPT, RPT)])


def _sc_layer(g_hbm, sd, pp, params, with_cnt):
    mesh = plsc.VectorSubcoreMesh(core_axis_name="c", subcore_axis_name="s")
    out_type = [jax.ShapeDtypeStruct((NC, NP, 2 * HID), jnp.float32)]
    scratch = [
        pltpu.VMEM_SHARED((NP, 2 * HID), jnp.float32),
    ]
    if with_cnt:
        out_type.append(jax.ShapeDtypeStruct((NC, 1, NP), jnp.float32))
        scratch.append(pltpu.VMEM_SHARED((NP,), jnp.float32))
    scratch += [
        pltpu.VMEM((2, CHUNK // 128, 2, 128), jnp.int32),
        pltpu.VMEM((2, 2, CHUNK), jnp.float32),
        pltpu.VMEM((2, CHUNK, 4 * HID), jnp.float32),
        pltpu.VMEM((CHUNK, 2 * HID), jnp.float32),
        pltpu.VMEM((2, CHUNK), jnp.float32),
        pltpu.VMEM((2, CHUNK), jnp.float32),
        pltpu.VMEM((128,), jnp.float32),
        pltpu.VMEM((128,), jnp.float32),
        [pltpu.SemaphoreType.DMA, pltpu.SemaphoreType.DMA],
    ]
    z2d = jnp.zeros((RPT, 2 * HID), jnp.float32)
    z1d = jnp.zeros((RPT,), jnp.float32)
    ones128 = jnp.ones((128,), jnp.float32)
    fn = pl.kernel(
        functools.partial(_sc_layer_body, with_cnt),
        out_type=out_type,
        mesh=mesh,
        scratch_types=scratch,
        compiler_params=pltpu.CompilerParams(use_tc_tiling_on_sc=False),
    )
    return fn(g_hbm, sd, pp, params, z2d, z1d, ones128)


def _elu(v):
    return jnp.where(v > 0, v, jnp.exp(v) - 1.0)


def _tc2_body(acc_ref, cnt_ref, root_ref, wg2_ref, wr2_ref, b2_ref,
              h1_ref, g2_ref, root2_ref):
    a = acc_ref[0] + acc_ref[1]
    cc = cnt_ref[0, 0, :] + cnt_ref[1, 0, :]
    inv = 1.0 / jnp.maximum(cc, 1.0)
    pre = a * inv[:, None] + root_ref[...]
    h1 = _elu(pre)
    h1_ref[...] = h1
    wg2 = wg2_ref[...]
    wr2 = wr2_ref[...]
    b2 = b2_ref[...]
    h1b0 = h1[:, 0:16]
    h1b1 = h1[:, 16:32]
    g2_ref[:, 0:32] = jnp.dot(h1b0, wg2, preferred_element_type=jnp.float32)
    g2_ref[:, 32:64] = jnp.dot(h1b1, wg2, preferred_element_type=jnp.float32)
    root2_ref[:, 0:16] = jnp.dot(h1b0, wr2, preferred_element_type=jnp.float32) + b2
    root2_ref[:, 16:32] = jnp.dot(h1b1, wr2, preferred_element_type=jnp.float32) + b2


def _tc2(acc1, cnt1, root1, Wg2, Wroot2, b2):
    return pl.pallas_call(
        _tc2_body,
        grid=(NBLK,),
        in_specs=[
            pl.BlockSpec((NC, BN, 2 * HID), lambda i: (0, i, 0)),
            pl.BlockSpec((NC, 1, BN), lambda i: (0, 0, i)),
            pl.BlockSpec((BN, 2 * HID), lambda i: (i, 0)),
            pl.BlockSpec((HID, K * HID), lambda i: (0, 0)),
            pl.BlockSpec((HID, HID), lambda i: (0, 0)),
            pl.BlockSpec((1, HID), lambda i: (0, 0)),
        ],
        out_specs=[
            pl.BlockSpec((BN, 2 * HID), lambda i: (i, 0)),
            pl.BlockSpec((BN, 2 * K * HID), lambda i: (i, 0)),
            pl.BlockSpec((BN, 2 * HID), lambda i: (i, 0)),
        ],
        out_shape=[
            jax.ShapeDtypeStruct((NP, 2 * HID), jnp.float32),
            jax.ShapeDtypeStruct((NP, 2 * K * HID), jnp.float32),
            jax.ShapeDtypeStruct((NP, 2 * HID), jnp.float32),
        ],
    )(acc1, cnt1, root1, Wg2, Wroot2, b2)


def _tc3_body(acc_ref, cnt_ref, root2_ref, h1_ref, wfce_ref, wfco_ref,
              bfc_ref, wl1_ref, bl1_ref, wl2_ref, bl2_ref, out_ref, zacc_ref):
    i = pl.program_id(0)
    a = acc_ref[0] + acc_ref[1]
    cc = cnt_ref[0, 0, :] + cnt_ref[1, 0, :]
    inv = 1.0 / jnp.maximum(cc, 1.0)
    h2 = _elu(a * inv[:, None] + root2_ref[...])
    h1 = h1_ref[...]
    wfce = wfce_ref[...]
    wfco = wfco_ref[...]
    s0 = (jnp.dot(h1[:, 0:16], wfce, preferred_element_type=jnp.float32)
          + jnp.dot(h2[:, 0:16], wfco, preferred_element_type=jnp.float32))
    s1 = (jnp.dot(h1[:, 16:32], wfce, preferred_element_type=jnp.float32)
          + jnp.dot(h2[:, 16:32], wfco, preferred_element_type=jnp.float32))
    sblk = jnp.concatenate([s0, s1], axis=1) + bfc_ref[...]
    contrib = lax.dot_general(sblk, wl1_ref[...],
                              (((0,), (0,)), ((), ())),
                              preferred_element_type=jnp.float32)

    @pl.when(i == 0)
    def _():
        zacc_ref[...] = jnp.zeros_like(zacc_ref)

    zacc_ref[...] += contrib

    @pl.when(i == NBLK - 1)
    def _():
        z = _elu(zacc_ref[...] + bl1_ref[...])
        zz = jnp.dot(z, wl2_ref[...], preferred_element_type=jnp.float32) + bl2_ref[...]
        m = jnp.max(zz, axis=-1, keepdims=True)
        lse = m + jnp.log(jnp.sum(jnp.exp(zz - m), axis=-1, keepdims=True))
        out_ref[...] = zz - lse


def _tc3(acc2, cnt1, root2, h1, wfce, wfco, bfc, Wl1p, bl1, Wl2, bl2):
    return pl.pallas_call(
        _tc3_body,
        grid=(NBLK,),
        in_specs=[
            pl.BlockSpec((NC, BN, 2 * HID), lambda i: (0, i, 0)),
            pl.BlockSpec((NC, 1, BN), lambda i: (0, 0, i)),
            pl.BlockSpec((BN, 2 * HID), lambda i: (i, 0)),
            pl.BlockSpec((BN, 2 * HID), lambda i: (i, 0)),
            pl.BlockSpec((HID, 1), lambda i: (0, 0)),
            pl.BlockSpec((HID, 1), lambda i: (0, 0)),
            pl.BlockSpec((1, 1), lambda i: (0, 0)),
            pl.BlockSpec((BN, HFC), lambda i: (i, 0)),
            pl.BlockSpec((1, HFC), lambda i: (0, 0)),
            pl.BlockSpec((HFC, NCLS), lambda i: (0, 0)),
            pl.BlockSpec((1, NCLS), lambda i: (0, 0)),
        ],
        out_specs=pl.BlockSpec((BS, NCLS), lambda i: (0, 0)),
        out_shape=jax.ShapeDtypeStruct((BS, NCLS), jnp.float32),
        scratch_shapes=[pltpu.VMEM((BS, HFC), jnp.float32)],
    )(acc2, cnt1, root2, h1, wfce, wfco, bfc, Wl1p, bl1, Wl2, bl2)


def kernel(x, batch, edge_index, pseudo, Wg1, mu1, sigma1, Wroot1, b1,
           Wg2, mu2, sigma2, Wroot2, b2, Wfc, bfc, Wl1, bl1, Wl2, bl2):
    f32 = jnp.float32
    xp = jnp.pad(x, ((0, 0), (0, NP - N_NODES), (0, 0)))
    src = edge_index[0]
    dst = edge_index[1]
    pad_e = E_PAD - E
    srcp = jnp.concatenate([src, jnp.full((pad_e,), N_NODES, jnp.int32)])
    dstp = jnp.concatenate([dst, jnp.full((pad_e,), N_NODES, jnp.int32)])
    sd = jnp.stack([srcp.reshape(EROWS, 128), dstp.reshape(EROWS, 128)], axis=1)
    pT = jnp.concatenate([pseudo.T, jnp.zeros((D, pad_e), f32)], axis=1)
    pp = pT.reshape(D, NCHUNKS, CHUNK).transpose(1, 0, 2)

    params1 = jnp.repeat(
        jnp.concatenate([mu1.reshape(-1), sigma1.reshape(-1)]), 16)
    params2 = jnp.repeat(
        jnp.concatenate([mu2.reshape(-1), sigma2.reshape(-1)]), 16)

    g1, root1 = _tc1(xp, Wg1, Wroot1, b1.reshape(1, HID))
    acc1, cnt1 = _sc_layer(g1, sd, pp, params1, with_cnt=True)
    h1, g2, root2 = _tc2(acc1, cnt1, root1, Wg2, Wroot2, b2.reshape(1, HID))
    (acc2,) = _sc_layer(g2, sd, pp, params2, with_cnt=False)

    wfce = Wfc[0::2, :]
    wfco = Wfc[1::2, :]
    Wl1p = jnp.pad(Wl1, ((0, NP - N_NODES), (0, 0)))
    out = _tc3(acc2, cnt1, root2, h1, wfce, wfco, bfc.reshape(1, 1),
               Wl1p, bl1.reshape(1, HFC), Wl2, bl2.reshape(1, NCLS))
    return out


# speedup vs baseline: 186.6552x; 1.1136x over previous
import functools

import jax
import jax.numpy as jnp
from jax import lax
from jax.experimental import pallas as pl
from jax.experimental.pallas import tpu as pltpu
from jax.experimental.pallas import tpu_sc as plsc

BS = 2
N_NODES = 15135
E = 484320
N_FEAT = 128
HID = 16
K = 2
D = 2
HFC = 256
NCLS = 2
EPS = 1e-15

NP = 16384
BN = 512
NBLK = NP // BN
RPT = NP // 16

NC = 2
NS = 16
CHUNK = 512
CPW = 30
EW = CHUNK * CPW
E_PAD = EW * NC * NS
EROWS = E_PAD // 128
NCHUNKS = E_PAD // CHUNK


def _tc1_body(x_ref, wg_ref, wr_ref, b_ref, g_ref, root_ref):
    wg = wg_ref[...]
    wr = wr_ref[...]
    b = b_ref[...]
    x0 = x_ref[0]
    x1 = x_ref[1]
    g_ref[:, 0:32] = jnp.dot(x0, wg, preferred_element_type=jnp.float32)
    g_ref[:, 32:64] = jnp.dot(x1, wg, preferred_element_type=jnp.float32)
    root_ref[:, 0:16] = jnp.dot(x0, wr, preferred_element_type=jnp.float32) + b
    root_ref[:, 16:32] = jnp.dot(x1, wr, preferred_element_type=jnp.float32) + b


def _tc1(xp, Wg1, Wroot1, b1):
    return pl.pallas_call(
        _tc1_body,
        grid=(NBLK,),
        in_specs=[
            pl.BlockSpec((BS, BN, N_FEAT), lambda i: (0, i, 0)),
            pl.BlockSpec((N_FEAT, K * HID), lambda i: (0, 0)),
            pl.BlockSpec((N_FEAT, HID), lambda i: (0, 0)),
            pl.BlockSpec((1, HID), lambda i: (0, 0)),
        ],
        out_specs=[
            pl.BlockSpec((BN, 2 * K * HID), lambda i: (i, 0)),
            pl.BlockSpec((BN, 2 * HID), lambda i: (i, 0)),
        ],
        out_shape=[
            jax.ShapeDtypeStruct((NP, 2 * K * HID), jnp.float32),
            jax.ShapeDtypeStruct((NP, 2 * HID), jnp.float32),
        ],
    )(xp, Wg1, Wroot1, b1)


def _sc_layer_body(with_cnt, g_hbm, sd_hbm, pp_hbm, params, z2d, z1d, ones_hbm,
                   *rest):
    if with_cnt:
        (out_acc, out_cnt, acc, cntacc, sd_v, pp_v, rows_v, msg_v,
         w0_v, w1_v, ones_v, params_v, gsem) = rest
    else:
        (out_acc, acc, sd_v, pp_v, rows_v, msg_v,
         w0_v, w1_v, ones_v, params_v, gsem) = rest
        out_cnt = cntacc = None

    c = lax.axis_index("c")
    s = lax.axis_index("s")
    wid = c * NS + s

    pltpu.sync_copy(z2d, acc.at[pl.ds(s * RPT, RPT)])
    if with_cnt:
        pltpu.sync_copy(z1d, cntacc.at[pl.ds(s * RPT, RPT)])

    pltpu.sync_copy(ones_hbm, ones_v)
    pltpu.sync_copy(params, params_v)

    plsc.subcore_barrier()

    m00 = params_v[pl.ds(0, 16)]
    m01 = params_v[pl.ds(16, 16)]
    m10 = params_v[pl.ds(32, 16)]
    m11 = params_v[pl.ds(48, 16)]
    s00 = params_v[pl.ds(64, 16)]
    s01 = params_v[pl.ds(80, 16)]
    s10 = params_v[pl.ds(96, 16)]
    s11 = params_v[pl.ds(112, 16)]
    c00 = -0.5 / (EPS + s00 * s00)
    c01 = -0.5 / (EPS + s01 * s01)
    c10 = -0.5 / (EPS + s10 * s10)
    c11 = -0.5 / (EPS + s11 * s11)

    base_r = wid * (EW // 128)
    base_c = wid * CPW

    def load_small(ch, slot):
        pltpu.sync_copy(sd_hbm.at[pl.ds(base_r + ch * (CHUNK // 128),
                                        CHUNK // 128)], sd_v.at[slot])
        pltpu.sync_copy(pp_hbm.at[base_c + ch], pp_v.at[slot])

    def start_gather(slot):
        for j in range(CHUNK // 128):
            pltpu.async_copy(g_hbm.at[sd_v.at[slot, j, 0]],
                             rows_v.at[slot, pl.ds(j * 128, 128)], gsem[slot])

    def process(slot):
        @plsc.parallel_loop(0, CHUNK // 16, 1, unroll=2)
        def wbody(kk):
            p0 = pp_v[slot, 0, pl.ds(kk * 16, 16)]
            p1 = pp_v[slot, 1, pl.ds(kk * 16, 16)]
            d00 = p0 - m00
            d01 = p1 - m01
            w0_v[slot, pl.ds(kk * 16, 16)] = jnp.exp(d00 * d00 * c00 + d01 * d01 * c01)
            d10 = p0 - m10
            d11 = p1 - m11
            w1_v[slot, pl.ds(kk * 16, 16)] = jnp.exp(d10 * d10 * c10 + d11 * d11 * c11)

        pltpu.make_async_copy(g_hbm.at[pl.ds(0, CHUNK)],
                              rows_v.at[slot], gsem[slot]).wait()

        @plsc.parallel_loop(0, CHUNK, 1, unroll=8)
        def ebody(e):
            bw0 = jnp.full((16,), w0_v[slot, pl.ds(e, 1)][0], jnp.float32)
            bw1 = jnp.full((16,), w1_v[slot, pl.ds(e, 1)][0], jnp.float32)
            r00 = rows_v[slot, e, pl.ds(0, 16)]
            r01 = rows_v[slot, e, pl.ds(16, 16)]
            r10 = rows_v[slot, e, pl.ds(32, 16)]
            r11 = rows_v[slot, e, pl.ds(48, 16)]
            msg_v[e, pl.ds(0, 16)] = r00 * bw0 + r01 * bw1
            msg_v[e, pl.ds(16, 16)] = r10 * bw0 + r11 * bw1

        for j in range(CHUNK // 128):
            pltpu.sync_copy(msg_v.at[pl.ds(j * 128, 128)],
                            acc.at[sd_v.at[slot, j, 1]], add=True)
            if with_cnt:
                pltpu.sync_copy(ones_v, cntacc.at[sd_v.at[slot, j, 1]],
                                add=True)

    load_small(0, 0)
    start_gather(0)

    def pair_body(i, carry):
        c0 = 2 * i
        load_small(c0 + 1, 1)
        start_gather(1)
        process(0)

        @pl.when(i < CPW // 2 - 1)
        def _():
            load_small(c0 + 2, 0)
            start_gather(0)

        process(1)
        return carry

    lax.fori_loop(0, CPW // 2, pair_body, 0)

    plsc.subcore_barrier()

    pltpu.sync_copy(acc.at[pl.ds(s * RPT, RPT)],
                    out_acc.at[c, pl.ds(s * RPT, RPT)])
    if with_cnt:
        pltpu.sync_copy(cntacc.at[pl.ds(s * RPT, RPT)],
                        out_cnt.at[c, 0, pl.ds(s * RPT, RPT)])


def _sc_layer(g_hbm, sd, pp, params, with_cnt):
    mesh = plsc.VectorSubcoreMesh(core_axis_name="c", subcore_axis_name="s")
    out_type = [jax.ShapeDtypeStruct((NC, NP, 2 * HID), jnp.float32)]
    scratch = [
        pltpu.VMEM_SHARED((NP, 2 * HID), jnp.float32),
    ]
    if with_cnt:
        out_type.append(jax.ShapeDtypeStruct((NC, 1, NP), jnp.float32))
        scratch.append(pltpu.VMEM_SHARED((NP,), jnp.float32))
    scratch += [
        pltpu.VMEM((2, CHUNK // 128, 2, 128), jnp.int32),
        pltpu.VMEM((2, 2, CHUNK), jnp.float32),
        pltpu.VMEM((2, CHUNK, 4 * HID), jnp.float32),
        pltpu.VMEM((CHUNK, 2 * HID), jnp.float32),
        pltpu.VMEM((2, CHUNK), jnp.float32),
        pltpu.VMEM((2, CHUNK), jnp.float32),
        pltpu.VMEM((128,), jnp.float32),
        pltpu.VMEM((128,), jnp.float32),
        [pltpu.SemaphoreType.DMA, pltpu.SemaphoreType.DMA],
    ]
    z2d = jnp.zeros((RPT, 2 * HID), jnp.float32)
    z1d = jnp.zeros((RPT,), jnp.float32)
    ones128 = jnp.ones((128,), jnp.float32)
    fn = pl.kernel(
        functools.partial(_sc_layer_body, with_cnt),
        out_type=out_type,
        mesh=mesh,
        scratch_types=scratch,
        compiler_params=pltpu.CompilerParams(use_tc_tiling_on_sc=False),
    )
    return fn(g_hbm, sd, pp, params, z2d, z1d, ones128)


def _elu(v):
    return jnp.where(v > 0, v, jnp.exp(v) - 1.0)


def _tc2_body(acc_ref, cnt_ref, root_ref, wg2_ref, wr2_ref, b2_ref,
              h1_ref, g2_ref, root2_ref):
    a = acc_ref[0] + acc_ref[1]
    cc = cnt_ref[0, 0, :] + cnt_ref[1, 0, :]
    inv = 1.0 / jnp.maximum(cc, 1.0)
    pre = a * inv[:, None] + root_ref[...]
    h1 = _elu(pre)
    h1_ref[...] = h1
    wg2 = wg2_ref[...]
    wr2 = wr2_ref[...]
    b2 = b2_ref[...]
    h1b0 = h1[:, 0:16]
    h1b1 = h1[:, 16:32]
    g2_ref[:, 0:32] = jnp.dot(h1b0, wg2, preferred_element_type=jnp.float32)
    g2_ref[:, 32:64] = jnp.dot(h1b1, wg2, preferred_element_type=jnp.float32)
    root2_ref[:, 0:16] = jnp.dot(h1b0, wr2, preferred_element_type=jnp.float32) + b2
    root2_ref[:, 16:32] = jnp.dot(h1b1, wr2, preferred_element_type=jnp.float32) + b2


def _tc2(acc1, cnt1, root1, Wg2, Wroot2, b2):
    return pl.pallas_call(
        _tc2_body,
        grid=(NBLK,),
        in_specs=[
            pl.BlockSpec((NC, BN, 2 * HID), lambda i: (0, i, 0)),
            pl.BlockSpec((NC, 1, BN), lambda i: (0, 0, i)),
            pl.BlockSpec((BN, 2 * HID), lambda i: (i, 0)),
            pl.BlockSpec((HID, K * HID), lambda i: (0, 0)),
            pl.BlockSpec((HID, HID), lambda i: (0, 0)),
            pl.BlockSpec((1, HID), lambda i: (0, 0)),
        ],
        out_specs=[
            pl.BlockSpec((BN, 2 * HID), lambda i: (i, 0)),
            pl.BlockSpec((BN, 2 * K * HID), lambda i: (i, 0)),
            pl.BlockSpec((BN, 2 * HID), lambda i: (i, 0)),
        ],
        out_shape=[
            jax.ShapeDtypeStruct((NP, 2 * HID), jnp.float32),
            jax.ShapeDtypeStruct((NP, 2 * K * HID), jnp.float32),
            jax.ShapeDtypeStruct((NP, 2 * HID), jnp.float32),
        ],
    )(acc1, cnt1, root1, Wg2, Wroot2, b2)


def _tc3_body(acc_ref, cnt_ref, root2_ref, h1_ref, wfce_ref, wfco_ref,
              bfc_ref, wl1_ref, bl1_ref, wl2_ref, bl2_ref, out_ref, zacc_ref):
    i = pl.program_id(0)
    a = acc_ref[0] + acc_ref[1]
    cc = cnt_ref[0, 0, :] + cnt_ref[1, 0, :]
    inv = 1.0 / jnp.maximum(cc, 1.0)
    h2 = _elu(a * inv[:, None] + root2_ref[...])
    h1 = h1_ref[...]
    wfce = wfce_ref[...]
    wfco = wfco_ref[...]
    s0 = (jnp.dot(h1[:, 0:16], wfce, preferred_element_type=jnp.float32)
          + jnp.dot(h2[:, 0:16], wfco, preferred_element_type=jnp.float32))
    s1 = (jnp.dot(h1[:, 16:32], wfce, preferred_element_type=jnp.float32)
          + jnp.dot(h2[:, 16:32], wfco, preferred_element_type=jnp.float32))
    sblk = jnp.concatenate([s0, s1], axis=1) + bfc_ref[...]
    contrib = lax.dot_general(sblk, wl1_ref[...],
                              (((0,), (0,)), ((), ())),
                              preferred_element_type=jnp.float32)

    @pl.when(i == 0)
    def _():
        zacc_ref[...] = jnp.zeros_like(zacc_ref)

    zacc_ref[...] += contrib

    @pl.when(i == NBLK - 1)
    def _():
        z = _elu(zacc_ref[...] + bl1_ref[...])
        zz = jnp.dot(z, wl2_ref[...], preferred_element_type=jnp.float32) + bl2_ref[...]
        m = jnp.max(zz, axis=-1, keepdims=True)
        lse = m + jnp.log(jnp.sum(jnp.exp(zz - m), axis=-1, keepdims=True))
        out_ref[...] = zz - lse


def _tc3(acc2, cnt1, root2, h1, wfce, wfco, bfc, Wl1p, bl1, Wl2, bl2):
    return pl.pallas_call(
        _tc3_body,
        grid=(NBLK,),
        in_specs=[
            pl.BlockSpec((NC, BN, 2 * HID), lambda i: (0, i, 0)),
            pl.BlockSpec((NC, 1, BN), lambda i: (0, 0, i)),
            pl.BlockSpec((BN, 2 * HID), lambda i: (i, 0)),
            pl.BlockSpec((BN, 2 * HID), lambda i: (i, 0)),
            pl.BlockSpec((HID, 1), lambda i: (0, 0)),
            pl.BlockSpec((HID, 1), lambda i: (0, 0)),
            pl.BlockSpec((1, 1), lambda i: (0, 0)),
            pl.BlockSpec((BN, HFC), lambda i: (i, 0)),
            pl.BlockSpec((1, HFC), lambda i: (0, 0)),
            pl.BlockSpec((HFC, NCLS), lambda i: (0, 0)),
            pl.BlockSpec((1, NCLS), lambda i: (0, 0)),
        ],
        out_specs=pl.BlockSpec((BS, NCLS), lambda i: (0, 0)),
        out_shape=jax.ShapeDtypeStruct((BS, NCLS), jnp.float32),
        scratch_shapes=[pltpu.VMEM((BS, HFC), jnp.float32)],
    )(acc2, cnt1, root2, h1, wfce, wfco, bfc, Wl1p, bl1, Wl2, bl2)


def kernel(x, batch, edge_index, pseudo, Wg1, mu1, sigma1, Wroot1, b1,
           Wg2, mu2, sigma2, Wroot2, b2, Wfc, bfc, Wl1, bl1, Wl2, bl2):
    f32 = jnp.float32
    xp = jnp.pad(x, ((0, 0), (0, NP - N_NODES), (0, 0)))
    src = edge_index[0]
    dst = edge_index[1]
    pad_e = E_PAD - E
    srcp = jnp.concatenate([src, jnp.full((pad_e,), N_NODES, jnp.int32)])
    dstp = jnp.concatenate([dst, jnp.full((pad_e,), N_NODES, jnp.int32)])
    sd = jnp.stack([srcp.reshape(EROWS, 128), dstp.reshape(EROWS, 128)], axis=1)
    pT = jnp.concatenate([pseudo.T, jnp.zeros((D, pad_e), f32)], axis=1)
    pp = pT.reshape(D, NCHUNKS, CHUNK).transpose(1, 0, 2)

    params1 = jnp.repeat(
        jnp.concatenate([mu1.reshape(-1), sigma1.reshape(-1)]), 16)
    params2 = jnp.repeat(
        jnp.concatenate([mu2.reshape(-1), sigma2.reshape(-1)]), 16)

    g1, root1 = _tc1(xp, Wg1, Wroot1, b1.reshape(1, HID))
    acc1, cnt1 = _sc_layer(g1, sd, pp, params1, with_cnt=True)
    h1, g2, root2 = _tc2(acc1, cnt1, root1, Wg2, Wroot2, b2.reshape(1, HID))
    (acc2,) = _sc_layer(g2, sd, pp, params2, with_cnt=False)

    wfce = Wfc[0::2, :]
    wfco = Wfc[1::2, :]
    Wl1p = jnp.pad(Wl1, ((0, NP - N_NODES), (0, 0)))
    out = _tc3(acc2, cnt1, root2, h1, wfce, wfco, bfc.reshape(1, 1),
               Wl1p, bl1.reshape(1, HFC), Wl2, bl2.reshape(1, NCLS))
    return out


# speedup vs baseline: 187.7658x; 1.0060x over previous
import functools

import jax
import jax.numpy as jnp
from jax import lax
from jax.experimental import pallas as pl
from jax.experimental.pallas import tpu as pltpu
from jax.experimental.pallas import tpu_sc as plsc

BS = 2
N_NODES = 15135
E = 484320
N_FEAT = 128
HID = 16
K = 2
D = 2
HFC = 256
NCLS = 2
EPS = 1e-15

NP = 16384
BN = 512
NBLK = NP // BN
RPT = NP // 16

NC = 2
NS = 16
CHUNK = 384
CPW = 40
EW = CHUNK * CPW
E_PAD = EW * NC * NS
EROWS = E_PAD // 128
NCHUNKS = E_PAD // CHUNK


def _tc1_body(x_ref, wg_ref, wr_ref, b_ref, g_ref, root_ref):
    wg = wg_ref[...]
    wr = wr_ref[...]
    b = b_ref[...]
    x0 = x_ref[0]
    x1 = x_ref[1]
    g_ref[:, 0:32] = jnp.dot(x0, wg, preferred_element_type=jnp.float32)
    g_ref[:, 32:64] = jnp.dot(x1, wg, preferred_element_type=jnp.float32)
    root_ref[:, 0:16] = jnp.dot(x0, wr, preferred_element_type=jnp.float32) + b
    root_ref[:, 16:32] = jnp.dot(x1, wr, preferred_element_type=jnp.float32) + b


def _tc1(xp, Wg1, Wroot1, b1):
    return pl.pallas_call(
        _tc1_body,
        grid=(NBLK,),
        in_specs=[
            pl.BlockSpec((BS, BN, N_FEAT), lambda i: (0, i, 0)),
            pl.BlockSpec((N_FEAT, K * HID), lambda i: (0, 0)),
            pl.BlockSpec((N_FEAT, HID), lambda i: (0, 0)),
            pl.BlockSpec((1, HID), lambda i: (0, 0)),
        ],
        out_specs=[
            pl.BlockSpec((BN, 2 * K * HID), lambda i: (i, 0)),
            pl.BlockSpec((BN, 2 * HID), lambda i: (i, 0)),
        ],
        out_shape=[
            jax.ShapeDtypeStruct((NP, 2 * K * HID), jnp.float32),
            jax.ShapeDtypeStruct((NP, 2 * HID), jnp.float32),
        ],
    )(xp, Wg1, Wroot1, b1)


def _sc_layer_body(with_cnt, g_hbm, comb_hbm, ps_hbm, params, z2d, z1d,
                   ones_hbm, *rest):
    if with_cnt:
        (out_acc, out_cnt, acc, cntacc, comb_v, ps_v, dstc_v, rows_v, msg_v,
         w0_v, w1_v, ones_v, params_v, gsem, isem, psem, ssem) = rest
    else:
        (out_acc, acc, comb_v, ps_v, dstc_v, rows_v, msg_v,
         w0_v, w1_v, ones_v, params_v, gsem, isem, psem, ssem) = rest
        out_cnt = cntacc = None

    c = lax.axis_index("c")
    s = lax.axis_index("s")
    wid = c * NS + s

    pltpu.sync_copy(z2d, acc.at[pl.ds(s * RPT, RPT)])
    if with_cnt:
        pltpu.sync_copy(z1d, cntacc.at[pl.ds(s * RPT, RPT)])

    pltpu.sync_copy(ones_hbm, ones_v)
    pltpu.sync_copy(params, params_v)

    plsc.subcore_barrier()

    m00 = params_v[pl.ds(0, 16)]
    m01 = params_v[pl.ds(16, 16)]
    m10 = params_v[pl.ds(32, 16)]
    m11 = params_v[pl.ds(48, 16)]
    s00 = params_v[pl.ds(64, 16)]
    s01 = params_v[pl.ds(80, 16)]
    s10 = params_v[pl.ds(96, 16)]
    s11 = params_v[pl.ds(112, 16)]
    c00 = -0.5 / (EPS + s00 * s00)
    c01 = -0.5 / (EPS + s01 * s01)
    c10 = -0.5 / (EPS + s10 * s10)
    c11 = -0.5 / (EPS + s11 * s11)

    base_c = wid * CPW
    NSUB = CHUNK // 128

    def drain_comb(slot):
        pltpu.make_async_copy(comb_hbm.at[0], comb_v.at[slot],
                              isem[slot]).wait()
        pltpu.make_async_copy(ps_hbm.at[0], ps_v.at[slot],
                              psem[slot]).wait()

    def start_gather(slot):
        for j in range(NSUB):
            pltpu.async_copy(g_hbm.at[comb_v.at[slot, pl.ds(j * 128, 128)]],
                             rows_v.at[slot, pl.ds(j * 128, 128)], gsem[slot])

    def process(c, slot):
        for j in range(NSUB):
            for t in range(8):
                dstc_v[slot, j, pl.ds(t * 16, 16)] = (
                    comb_v[slot, pl.ds(CHUNK + j * 128 + t * 16, 16)])

        @plsc.parallel_loop(0, CHUNK // 16, 1, unroll=2)
        def wbody(kk):
            p0 = ps_v[slot, pl.ds(kk * 16, 16)]
            p1 = ps_v[slot, pl.ds(CHUNK + kk * 16, 16)]
            d00 = p0 - m00
            d01 = p1 - m01
            w0_v[slot, pl.ds(kk * 16, 16)] = jnp.exp(d00 * d00 * c00 + d01 * d01 * c01)
            d10 = p0 - m10
            d11 = p1 - m11
            w1_v[slot, pl.ds(kk * 16, 16)] = jnp.exp(d10 * d10 * c10 + d11 * d11 * c11)

        cn = jnp.minimum(c + 2, CPW - 1)
        pltpu.async_copy(ps_hbm.at[base_c + cn], ps_v.at[slot], psem[slot])

        pltpu.make_async_copy(g_hbm.at[pl.ds(0, CHUNK)],
                              rows_v.at[slot], gsem[slot]).wait()
        pltpu.async_copy(comb_hbm.at[base_c + cn], comb_v.at[slot], isem[slot])

        @pl.when(c >= 2)
        def _():
            for j in range(NSUB):
                pltpu.make_async_copy(msg_v.at[slot, pl.ds(0, 128)],
                                      acc.at[pl.ds(0, 128)], ssem[slot]).wait()
                if with_cnt:
                    pltpu.make_async_copy(ones_v, cntacc.at[pl.ds(0, 128)],
                                          ssem[slot]).wait()

        @plsc.parallel_loop(0, CHUNK, 1, unroll=8)
        def ebody(e):
            bw0 = jnp.full((16,), w0_v[slot, pl.ds(e, 1)][0], jnp.float32)
            bw1 = jnp.full((16,), w1_v[slot, pl.ds(e, 1)][0], jnp.float32)
            r00 = rows_v[slot, e, pl.ds(0, 16)]
            r01 = rows_v[slot, e, pl.ds(16, 16)]
            r10 = rows_v[slot, e, pl.ds(32, 16)]
            r11 = rows_v[slot, e, pl.ds(48, 16)]
            msg_v[slot, e, pl.ds(0, 16)] = r00 * bw0 + r01 * bw1
            msg_v[slot, e, pl.ds(16, 16)] = r10 * bw0 + r11 * bw1

        for j in range(NSUB):
            pltpu.async_copy(msg_v.at[slot, pl.ds(j * 128, 128)],
                             acc.at[dstc_v.at[slot, j]], ssem[slot], add=True)
            if with_cnt:
                pltpu.async_copy(ones_v, cntacc.at[dstc_v.at[slot, j]],
                                 ssem[slot], add=True)

    pltpu.async_copy(comb_hbm.at[base_c + 0], comb_v.at[0], isem[0])
    pltpu.async_copy(comb_hbm.at[base_c + 1], comb_v.at[1], isem[1])
    pltpu.async_copy(ps_hbm.at[base_c + 0], ps_v.at[0], psem[0])
    pltpu.async_copy(ps_hbm.at[base_c + 1], ps_v.at[1], psem[1])
    drain_comb(0)
    start_gather(0)

    def pair_body(i, carry):
        c0 = 2 * i
        drain_comb(1)
        start_gather(1)
        process(c0, 0)

        @pl.when(i < CPW // 2 - 1)
        def _():
            drain_comb(0)
            start_gather(0)

        process(c0 + 1, 1)
        return carry

    lax.fori_loop(0, CPW // 2, pair_body, 0)

    for slot in (0, 1):
        drain_comb(slot)
        for j in range(NSUB):
            pltpu.make_async_copy(msg_v.at[slot, pl.ds(0, 128)],
                                  acc.at[pl.ds(0, 128)], ssem[slot]).wait()
            if with_cnt:
                pltpu.make_async_copy(ones_v, cntacc.at[pl.ds(0, 128)],
                                      ssem[slot]).wait()

    plsc.subcore_barrier()

    pltpu.sync_copy(acc.at[pl.ds(s * RPT, RPT)],
                    out_acc.at[c, pl.ds(s * RPT, RPT)])
    if with_cnt:
        pltpu.sync_copy(cntacc.at[pl.ds(s * RPT, RPT)],
                        out_cnt.at[c, 0, pl.ds(s * RPT, RPT)])


def _sc_layer(g_hbm, comb, ps, params, with_cnt):
    mesh = plsc.VectorSubcoreMesh(core_axis_name="c", subcore_axis_name="s")
    out_type = [jax.ShapeDtypeStruct((NC, NP, 2 * HID), jnp.float32)]
    scratch = [
        pltpu.VMEM_SHARED((NP, 2 * HID), jnp.float32),
    ]
    if with_cnt:
        out_type.append(jax.ShapeDtypeStruct((NC, 1, NP), jnp.float32))
        scratch.append(pltpu.VMEM_SHARED((NP,), jnp.float32))
    scratch += [
        pltpu.VMEM((2, 2 * CHUNK), jnp.int32),
        pltpu.VMEM((2, 2 * CHUNK), jnp.float32),
        pltpu.VMEM((2, CHUNK // 128, 128), jnp.int32),
        pltpu.VMEM((2, CHUNK, 4 * HID), jnp.float32),
        pltpu.VMEM((2, CHUNK, 2 * HID), jnp.float32),
        pltpu.VMEM((2, CHUNK), jnp.float32),
        pltpu.VMEM((2, CHUNK), jnp.float32),
        pltpu.VMEM((128,), jnp.float32),
        pltpu.VMEM((128,), jnp.float32),
        [pltpu.SemaphoreType.DMA, pltpu.SemaphoreType.DMA],
        [pltpu.SemaphoreType.DMA, pltpu.SemaphoreType.DMA],
        [pltpu.SemaphoreType.DMA, pltpu.SemaphoreType.DMA],
        [pltpu.SemaphoreType.DMA, pltpu.SemaphoreType.DMA],
    ]
    z2d = jnp.zeros((RPT, 2 * HID), jnp.float32)
    z1d = jnp.zeros((RPT,), jnp.float32)
    ones128 = jnp.ones((128,), jnp.float32)
    fn = pl.kernel(
        functools.partial(_sc_layer_body, with_cnt),
        out_type=out_type,
        mesh=mesh,
        scratch_types=scratch,
        compiler_params=pltpu.CompilerParams(use_tc_tiling_on_sc=False),
    )
    return fn(g_hbm, comb, ps, params, z2d, z1d, ones128)


def _elu(v):
    return jnp.where(v > 0, v, jnp.exp(v) - 1.0)


def _tc2_body(acc_ref, cnt_ref, root_ref, wg2_ref, wr2_ref, b2_ref,
              h1_ref, g2_ref, root2_ref):
    a = acc_ref[0] + acc_ref[1]
    cc = cnt_ref[0, 0, :] + cnt_ref[1, 0, :]
    inv = 1.0 / jnp.maximum(cc, 1.0)
    pre = a * inv[:, None] + root_ref[...]
    h1 = _elu(pre)
    h1_ref[...] = h1
    wg2 = wg2_ref[...]
    wr2 = wr2_ref[...]
    b2 = b2_ref[...]
    h1b0 = h1[:, 0:16]
    h1b1 = h1[:, 16:32]
    g2_ref[:, 0:32] = jnp.dot(h1b0, wg2, preferred_element_type=jnp.float32)
    g2_ref[:, 32:64] = jnp.dot(h1b1, wg2, preferred_element_type=jnp.float32)
    root2_ref[:, 0:16] = jnp.dot(h1b0, wr2, preferred_element_type=jnp.float32) + b2
    root2_ref[:, 16:32] = jnp.dot(h1b1, wr2, preferred_element_type=jnp.float32) + b2


def _tc2(acc1, cnt1, root1, Wg2, Wroot2, b2):
    return pl.pallas_call(
        _tc2_body,
        grid=(NBLK,),
        in_specs=[
            pl.BlockSpec((NC, BN, 2 * HID), lambda i: (0, i, 0)),
            pl.BlockSpec((NC, 1, BN), lambda i: (0, 0, i)),
            pl.BlockSpec((BN, 2 * HID), lambda i: (i, 0)),
            pl.BlockSpec((HID, K * HID), lambda i: (0, 0)),
            pl.BlockSpec((HID, HID), lambda i: (0, 0)),
            pl.BlockSpec((1, HID), lambda i: (0, 0)),
        ],
        out_specs=[
            pl.BlockSpec((BN, 2 * HID), lambda i: (i, 0)),
            pl.BlockSpec((BN, 2 * K * HID), lambda i: (i, 0)),
            pl.BlockSpec((BN, 2 * HID), lambda i: (i, 0)),
        ],
        out_shape=[
            jax.ShapeDtypeStruct((NP, 2 * HID), jnp.float32),
            jax.ShapeDtypeStruct((NP, 2 * K * HID), jnp.float32),
            jax.ShapeDtypeStruct((NP, 2 * HID), jnp.float32),
        ],
    )(acc1, cnt1, root1, Wg2, Wroot2, b2)


def _tc3_body(acc_ref, cnt_ref, root2_ref, h1_ref, wfce_ref, wfco_ref,
              bfc_ref, wl1_ref, bl1_ref, wl2_ref, bl2_ref, out_ref, zacc_ref):
    i = pl.program_id(0)
    a = acc_ref[0] + acc_ref[1]
    cc = cnt_ref[0, 0, :] + cnt_ref[1, 0, :]
    inv = 1.0 / jnp.maximum(cc, 1.0)
    h2 = _elu(a * inv[:, None] + root2_ref[...])
    h1 = h1_ref[...]
    wfce = wfce_ref[...]
    wfco = wfco_ref[...]
    s0 = (jnp.dot(h1[:, 0:16], wfce, preferred_element_type=jnp.float32)
          + jnp.dot(h2[:, 0:16], wfco, preferred_element_type=jnp.float32))
    s1 = (jnp.dot(h1[:, 16:32], wfce, preferred_element_type=jnp.float32)
          + jnp.dot(h2[:, 16:32], wfco, preferred_element_type=jnp.float32))
    sblk = jnp.concatenate([s0, s1], axis=1) + bfc_ref[...]
    contrib = lax.dot_general(sblk, wl1_ref[...],
                              (((0,), (0,)), ((), ())),
                              preferred_element_type=jnp.float32)

    @pl.when(i == 0)
    def _():
        zacc_ref[...] = jnp.zeros_like(zacc_ref)

    zacc_ref[...] += contrib

    @pl.when(i == NBLK - 1)
    def _():
        z = _elu(zacc_ref[...] + bl1_ref[...])
        zz = jnp.dot(z, wl2_ref[...], preferred_element_type=jnp.float32) + bl2_ref[...]
        m = jnp.max(zz, axis=-1, keepdims=True)
        lse = m + jnp.log(jnp.sum(jnp.exp(zz - m), axis=-1, keepdims=True))
        out_ref[...] = zz - lse


def _tc3(acc2, cnt1, root2, h1, wfce, wfco, bfc, Wl1p, bl1, Wl2, bl2):
    return pl.pallas_call(
        _tc3_body,
        grid=(NBLK,),
        in_specs=[
            pl.BlockSpec((NC, BN, 2 * HID), lambda i: (0, i, 0)),
            pl.BlockSpec((NC, 1, BN), lambda i: (0, 0, i)),
            pl.BlockSpec((BN, 2 * HID), lambda i: (i, 0)),
            pl.BlockSpec((BN, 2 * HID), lambda i: (i, 0)),
            pl.BlockSpec((HID, 1), lambda i: (0, 0)),
            pl.BlockSpec((HID, 1), lambda i: (0, 0)),
            pl.BlockSpec((1, 1), lambda i: (0, 0)),
            pl.BlockSpec((BN, HFC), lambda i: (i, 0)),
            pl.BlockSpec((1, HFC), lambda i: (0, 0)),
            pl.BlockSpec((HFC, NCLS), lambda i: (0, 0)),
            pl.BlockSpec((1, NCLS), lambda i: (0, 0)),
        ],
        out_specs=pl.BlockSpec((BS, NCLS), lambda i: (0, 0)),
        out_shape=jax.ShapeDtypeStruct((BS, NCLS), jnp.float32),
        scratch_shapes=[pltpu.VMEM((BS, HFC), jnp.float32)],
    )(acc2, cnt1, root2, h1, wfce, wfco, bfc, Wl1p, bl1, Wl2, bl2)


def kernel(x, batch, edge_index, pseudo, Wg1, mu1, sigma1, Wroot1, b1,
           Wg2, mu2, sigma2, Wroot2, b2, Wfc, bfc, Wl1, bl1, Wl2, bl2):
    f32 = jnp.float32
    xp = jnp.pad(x, ((0, 0), (0, NP - N_NODES), (0, 0)))
    src = edge_index[0]
    dst = edge_index[1]
    pad_e = E_PAD - E
    srcp = jnp.concatenate([src, jnp.full((pad_e,), N_NODES, jnp.int32)])
    dstp = jnp.concatenate([dst, jnp.full((pad_e,), N_NODES, jnp.int32)])
    pT = jnp.concatenate([pseudo.T, jnp.zeros((D, pad_e), f32)], axis=1)
    comb = jnp.concatenate(
        [srcp.reshape(NCHUNKS, 1, CHUNK), dstp.reshape(NCHUNKS, 1, CHUNK)],
        axis=1).reshape(NCHUNKS, 2 * CHUNK)
    ps = jnp.concatenate(
        [pT[0].reshape(NCHUNKS, 1, CHUNK), pT[1].reshape(NCHUNKS, 1, CHUNK)],
        axis=1).reshape(NCHUNKS, 2 * CHUNK)

    params1 = jnp.repeat(
        jnp.concatenate([mu1.reshape(-1), sigma1.reshape(-1)]), 16)
    params2 = jnp.repeat(
        jnp.concatenate([mu2.reshape(-1), sigma2.reshape(-1)]), 16)

    g1, root1 = _tc1(xp, Wg1, Wroot1, b1.reshape(1, HID))
    acc1, cnt1 = _sc_layer(g1, comb, ps, params1, with_cnt=True)
    h1, g2, root2 = _tc2(acc1, cnt1, root1, Wg2, Wroot2, b2.reshape(1, HID))
    (acc2,) = _sc_layer(g2, comb, ps, params2, with_cnt=False)

    wfce = Wfc[0::2, :]
    wfco = Wfc[1::2, :]
    Wl1p = jnp.pad(Wl1, ((0, NP - N_NODES), (0, 0)))
    out = _tc3(acc2, cnt1, root2, h1, wfce, wfco, bfc.reshape(1, 1),
               Wl1p, bl1.reshape(1, HFC), Wl2, bl2.reshape(1, NCLS))
    return out
